# Initial kernel scaffold; baseline (speedup 1.0000x reference)
import jax
import jax.numpy as jnp
from jax.experimental import pallas as pl


def kernel(x, router_w, router_b, lora_A, lora_B):
    raise NotImplementedError("write your pallas kernel here")


# speedup vs baseline: 3.0534x; 3.0534x over previous
import jax
import jax.numpy as jnp
from jax.experimental import pallas as pl


def _moe_block_kernel(x_ref, rw_ref, rb_ref, a_ref, b_ref, out_ref):
    x = x_ref[...]
    n_exp = rw_ref.shape[0]
    rank = a_ref.shape[1]
    scaling = 1.0 / rank
    logits = jax.lax.dot_general(
        x, rw_ref[...], (((1,), (1,)), ((), ())),
        preferred_element_type=jnp.float32) + rb_ref[...]
    m = jnp.max(logits, axis=1, keepdims=True)
    p = jnp.exp(logits - m)
    probs = p / jnp.sum(p, axis=1, keepdims=True)
    pmax = jnp.max(probs, axis=1, keepdims=True)
    iota = jax.lax.broadcasted_iota(jnp.int32, probs.shape, 1)
    idx = jnp.min(jnp.where(probs == pmax, iota, n_exp), axis=1, keepdims=True)
    acc = jnp.zeros(out_ref.shape, out_ref.dtype)
    for e in range(n_exp):
        h = jax.lax.dot_general(
            x, a_ref[e], (((1,), (1,)), ((), ())),
            preferred_element_type=jnp.float32)
        o = jax.lax.dot_general(
            h, b_ref[e], (((1,), (1,)), ((), ())),
            preferred_element_type=jnp.float32) * scaling
        acc = jnp.where(idx == e, o, acc)
    out_ref[...] = acc * pmax


def kernel(x, router_w, router_b, lora_A, lora_B):
    b, s, d = x.shape
    n = b * s
    n_exp, rank, _ = lora_A.shape
    out_dim = lora_B.shape[1]
    x_flat = x.reshape(n, d)
    tb = 512 if n % 512 == 0 else n
    out = pl.pallas_call(
        _moe_block_kernel,
        grid=(n // tb,),
        in_specs=[
            pl.BlockSpec((tb, d), lambda i: (i, 0)),
            pl.BlockSpec((n_exp, d), lambda i: (0, 0)),
            pl.BlockSpec((1, n_exp), lambda i: (0, 0)),
            pl.BlockSpec((n_exp, rank, d), lambda i: (0, 0, 0)),
            pl.BlockSpec((n_exp, out_dim, rank), lambda i: (0, 0, 0)),
        ],
        out_specs=pl.BlockSpec((tb, out_dim), lambda i: (i, 0)),
        out_shape=jax.ShapeDtypeStruct((n, out_dim), x.dtype),
    )(x_flat, router_w, router_b.reshape(1, n_exp), lora_A, lora_B)
    return out.reshape(b, s, out_dim)


# speedup vs baseline: 7.3141x; 2.3954x over previous
import jax
import jax.numpy as jnp
from jax.experimental import pallas as pl


def _moe_block_kernel(x_ref, rw_ref, rb_ref, a_ref, b_ref, out_ref, *, rank):
    x = x_ref[...]
    n_exp = rw_ref.shape[0]
    scaling = 1.0 / rank
    logits = jax.lax.dot_general(
        x, rw_ref[...], (((1,), (1,)), ((), ())),
        preferred_element_type=jnp.float32) + rb_ref[...]
    m = jnp.max(logits, axis=1, keepdims=True)
    p = jnp.exp(logits - m)
    probs = p / jnp.sum(p, axis=1, keepdims=True)
    pmax = jnp.max(probs, axis=1, keepdims=True)
    iota = jax.lax.broadcasted_iota(jnp.int32, probs.shape, 1)
    idx = jnp.min(jnp.where(probs == pmax, iota, n_exp), axis=1, keepdims=True)
    xb = x.astype(jnp.bfloat16)
    h = jax.lax.dot_general(
        xb, a_ref[...], (((1,), (1,)), ((), ())),
        preferred_element_type=jnp.float32)
    col_e = jax.lax.broadcasted_iota(jnp.int32, h.shape, 1) // rank
    hm = jnp.where(col_e == idx, h, 0.0) * (pmax * scaling)
    o = jax.lax.dot_general(
        hm.astype(jnp.bfloat16), b_ref[...], (((1,), (1,)), ((), ())),
        preferred_element_type=jnp.float32)
    out_ref[...] = o


def kernel(x, router_w, router_b, lora_A, lora_B):
    import functools
    b, s, d = x.shape
    n = b * s
    n_exp, rank, _ = lora_A.shape
    out_dim = lora_B.shape[1]
    er = n_exp * rank
    x_flat = x.reshape(n, d)
    a_all = lora_A.reshape(er, d).astype(jnp.bfloat16)
    b_all = jnp.swapaxes(lora_B, 0, 1).reshape(out_dim, er).astype(jnp.bfloat16)
    tb = 512 if n % 512 == 0 else n
    out = pl.pallas_call(
        functools.partial(_moe_block_kernel, rank=rank),
        grid=(n // tb,),
        in_specs=[
            pl.BlockSpec((tb, d), lambda i: (i, 0)),
            pl.BlockSpec((n_exp, d), lambda i: (0, 0)),
            pl.BlockSpec((1, n_exp), lambda i: (0, 0)),
            pl.BlockSpec((er, d), lambda i: (0, 0)),
            pl.BlockSpec((out_dim, er), lambda i: (0, 0)),
        ],
        out_specs=pl.BlockSpec((tb, out_dim), lambda i: (i, 0)),
        out_shape=jax.ShapeDtypeStruct((n, out_dim), x.dtype),
    )(x_flat, router_w, router_b.reshape(1, n_exp), a_all, b_all)
    return out.reshape(b, s, out_dim)


# speedup vs baseline: 7.3699x; 1.0076x over previous
import jax
import jax.numpy as jnp
from jax.experimental import pallas as pl


def _moe_block_kernel(x_ref, rw_ref, rb_ref, a_ref, b_ref, out_ref, *, rank):
    x = x_ref[...]
    n_exp = rw_ref.shape[0]
    scaling = 1.0 / rank
    logits = jax.lax.dot_general(
        x, rw_ref[...], (((1,), (1,)), ((), ())),
        preferred_element_type=jnp.float32) + rb_ref[...]
    m = jnp.max(logits, axis=1, keepdims=True)
    p = jnp.exp(logits - m)
    probs = p / jnp.sum(p, axis=1, keepdims=True)
    pmax = jnp.max(probs, axis=1, keepdims=True)
    iota = jax.lax.broadcasted_iota(jnp.int32, probs.shape, 1)
    idx = jnp.min(jnp.where(probs == pmax, iota, n_exp), axis=1, keepdims=True)
    xb = x.astype(jnp.bfloat16)
    h = jax.lax.dot_general(
        xb, a_ref[...], (((1,), (1,)), ((), ())),
        preferred_element_type=jnp.float32)
    col_e = jax.lax.broadcasted_iota(jnp.int32, h.shape, 1) // rank
    hm = jnp.where(col_e == idx, h, 0.0) * (pmax * scaling)
    o = jax.lax.dot_general(
        hm.astype(jnp.bfloat16), b_ref[...], (((1,), (1,)), ((), ())),
        preferred_element_type=jnp.float32)
    out_ref[...] = o


def kernel(x, router_w, router_b, lora_A, lora_B):
    import functools
    b, s, d = x.shape
    n = b * s
    n_exp, rank, _ = lora_A.shape
    out_dim = lora_B.shape[1]
    er = n_exp * rank
    x_flat = x.reshape(n, d)
    a_all = lora_A.reshape(er, d).astype(jnp.bfloat16)
    b_all = jnp.swapaxes(lora_B, 0, 1).reshape(out_dim, er).astype(jnp.bfloat16)
    tb = 1024 if n % 1024 == 0 else n
    out = pl.pallas_call(
        functools.partial(_moe_block_kernel, rank=rank),
        grid=(n // tb,),
        in_specs=[
            pl.BlockSpec((tb, d), lambda i: (i, 0)),
            pl.BlockSpec((n_exp, d), lambda i: (0, 0)),
            pl.BlockSpec((1, n_exp), lambda i: (0, 0)),
            pl.BlockSpec((er, d), lambda i: (0, 0)),
            pl.BlockSpec((out_dim, er), lambda i: (0, 0)),
        ],
        out_specs=pl.BlockSpec((tb, out_dim), lambda i: (i, 0)),
        out_shape=jax.ShapeDtypeStruct((n, out_dim), x.dtype),
    )(x_flat, router_w, router_b.reshape(1, n_exp), a_all, b_all)
    return out.reshape(b, s, out_dim)


# speedup vs baseline: 7.8434x; 1.0643x over previous
import functools

import jax
import jax.numpy as jnp
from jax.experimental import pallas as pl


def _moe_block_kernel(cm_ref, x_ref, rw_ref, rb_ref, a_ref, b_ref, out_ref,
                      *, rank, halves):
    n_exp = rw_ref.shape[0]
    scaling = 1.0 / rank
    hb = x_ref.shape[0] // halves
    for c in range(halves):
        rows = pl.ds(c * hb, hb)
        x = x_ref[rows, :]
        logits = jax.lax.dot_general(
            x, rw_ref[...], (((1,), (1,)), ((), ())),
            preferred_element_type=jnp.float32) + rb_ref[...]
        m = jnp.max(logits, axis=1, keepdims=True)
        p = jnp.exp(logits - m)
        probs = p / jnp.sum(p, axis=1, keepdims=True)
        pmax = jnp.max(probs, axis=1, keepdims=True)
        iota = jax.lax.broadcasted_iota(jnp.int32, probs.shape, 1)
        idx = jnp.min(jnp.where(probs == pmax, iota, n_exp), axis=1,
                      keepdims=True)
        h = jax.lax.dot_general(
            x.astype(jnp.bfloat16), a_ref[...], (((1,), (1,)), ((), ())),
            preferred_element_type=jnp.float32)
        gate = jnp.where(cm_ref[...] == idx, pmax * scaling, 0.0)
        o = jax.lax.dot_general(
            (h * gate).astype(jnp.bfloat16), b_ref[...],
            (((1,), (1,)), ((), ())), preferred_element_type=jnp.float32)
        out_ref[rows, :] = o


def kernel(x, router_w, router_b, lora_A, lora_B):
    b, s, d = x.shape
    n = b * s
    n_exp, rank, _ = lora_A.shape
    out_dim = lora_B.shape[1]
    er = n_exp * rank
    x_flat = x.reshape(n, d)
    a_all = lora_A.reshape(er, d).astype(jnp.bfloat16)
    b_all = jnp.swapaxes(lora_B, 0, 1).reshape(out_dim, er).astype(jnp.bfloat16)
    colmap = (jnp.arange(er, dtype=jnp.int32) // rank).reshape(1, er)
    tb = 1024 if n % 1024 == 0 else n
    halves = 2 if tb % 2 == 0 else 1
    out = pl.pallas_call(
        functools.partial(_moe_block_kernel, rank=rank, halves=halves),
        grid=(n // tb,),
        in_specs=[
            pl.BlockSpec((1, er), lambda i: (0, 0)),
            pl.BlockSpec((tb, d), lambda i: (i, 0)),
            pl.BlockSpec((n_exp, d), lambda i: (0, 0)),
            pl.BlockSpec((1, n_exp), lambda i: (0, 0)),
            pl.BlockSpec((er, d), lambda i: (0, 0)),
            pl.BlockSpec((out_dim, er), lambda i: (0, 0)),
        ],
        out_specs=pl.BlockSpec((tb, out_dim), lambda i: (i, 0)),
        out_shape=jax.ShapeDtypeStruct((n, out_dim), x.dtype),
    )(colmap, x_flat, router_w, router_b.reshape(1, n_exp), a_all, b_all)
    return out.reshape(b, s, out_dim)


# speedup vs baseline: 8.0385x; 1.0249x over previous
import functools

import jax
import jax.numpy as jnp
from jax.experimental import pallas as pl


def _moe_block_kernel(cm_ref, x_ref, rw_ref, rb_ref, a_ref, b_ref, out_ref,
                      *, rank, halves):
    n_exp = rw_ref.shape[0]
    scaling = 1.0 / rank
    hb = x_ref.shape[0] // halves
    for c in range(halves):
        rows = pl.ds(c * hb, hb)
        x = x_ref[rows, :]
        logits = jax.lax.dot_general(
            x, rw_ref[...], (((1,), (1,)), ((), ())),
            preferred_element_type=jnp.float32) + rb_ref[...]
        m = jnp.max(logits, axis=1, keepdims=True)
        p = jnp.exp(logits - m)
        probs = p / jnp.sum(p, axis=1, keepdims=True)
        pmax = jnp.max(probs, axis=1, keepdims=True)
        iota = jax.lax.broadcasted_iota(jnp.int32, probs.shape, 1)
        idx = jnp.min(jnp.where(probs == pmax, iota, n_exp), axis=1,
                      keepdims=True)
        h = jax.lax.dot_general(
            x.astype(jnp.bfloat16), a_ref[...], (((1,), (1,)), ((), ())),
            preferred_element_type=jnp.float32)
        gate = jnp.where(cm_ref[...] == idx, pmax * scaling, 0.0)
        o = jax.lax.dot_general(
            (h * gate).astype(jnp.bfloat16), b_ref[...],
            (((1,), (1,)), ((), ())), preferred_element_type=jnp.float32)
        out_ref[rows, :] = o


def kernel(x, router_w, router_b, lora_A, lora_B):
    b, s, d = x.shape
    n = b * s
    n_exp, rank, _ = lora_A.shape
    out_dim = lora_B.shape[1]
    er = n_exp * rank
    x_flat = x.reshape(n, d)
    a_all = lora_A.reshape(er, d).astype(jnp.bfloat16)
    b_all = jnp.swapaxes(lora_B, 0, 1).reshape(out_dim, er).astype(jnp.bfloat16)
    colmap = (jnp.arange(er, dtype=jnp.int32) // rank).reshape(1, er)
    tb = 1024 if n % 1024 == 0 else n
    halves = 4 if tb % 4 == 0 else 1
    out = pl.pallas_call(
        functools.partial(_moe_block_kernel, rank=rank, halves=halves),
        grid=(n // tb,),
        in_specs=[
            pl.BlockSpec((1, er), lambda i: (0, 0)),
            pl.BlockSpec((tb, d), lambda i: (i, 0)),
            pl.BlockSpec((n_exp, d), lambda i: (0, 0)),
            pl.BlockSpec((1, n_exp), lambda i: (0, 0)),
            pl.BlockSpec((er, d), lambda i: (0, 0)),
            pl.BlockSpec((out_dim, er), lambda i: (0, 0)),
        ],
        out_specs=pl.BlockSpec((tb, out_dim), lambda i: (i, 0)),
        out_shape=jax.ShapeDtypeStruct((n, out_dim), x.dtype),
    )(colmap, x_flat, router_w, router_b.reshape(1, n_exp), a_all, b_all)
    return out.reshape(b, s, out_dim)
